# baseline (speedup 1.0000x reference)
import functools

import jax
import jax.numpy as jnp
from jax import lax
from jax.experimental import pallas as pl
from jax.experimental.pallas import tpu as pltpu

_N = 10000
_H = 8
_C = 128
_E = 320000
_EP = _E + _N
_CH = 3000
_NB = _EP // _CH
_BN = 1000
_TH = 3336
_NT = 3

_VMEM_LIM = 100 * 1024 * 1024


def _leaky(v, slope):
    return jnp.where(v >= 0, v, slope * v)


def _k1(x_ref, w_ref, asm_ref, adm_ref, h_ref, asad_ref, asm_o, adm_o):
    i = pl.program_id(0)
    h = jnp.dot(x_ref[...], w_ref[...], preferred_element_type=jnp.float32, precision=jax.lax.Precision.HIGHEST)
    h_ref[...] = h
    a_s = jnp.dot(h, asm_ref[...], preferred_element_type=jnp.float32, precision=jax.lax.Precision.HIGHEST)
    a_d = jnp.dot(h, adm_ref[...], preferred_element_type=jnp.float32, precision=jax.lax.Precision.HIGHEST)
    asad_ref[...] = jnp.concatenate([a_s, a_d], axis=1)
    bmax_s = jnp.max(a_s, axis=0, keepdims=True)
    bmax_d = jnp.max(a_d, axis=0, keepdims=True)

    @pl.when(i == 0)
    def _():
        asm_o[...] = bmax_s
        adm_o[...] = bmax_d

    @pl.when(i > 0)
    def _():
        asm_o[...] = jnp.maximum(asm_o[...], bmax_s)
        adm_o[...] = jnp.maximum(adm_o[...], bmax_d)


def _k2(t, src_ref, dst_ref, h_ref, asad_ref, m1_ref, u_ref, den_ref):
    @pl.when(pl.program_id(0) == 0)
    def _():
        u_ref[...] = jnp.zeros_like(u_ref)
        den_ref[...] = jnp.zeros_like(den_ref)

    m1 = m1_ref[...]
    base = t * _TH

    def body(i, _):
        s = src_ref[0, 0, i]
        d = dst_ref[0, 0, i]
        inr = jnp.logical_and(d >= base, d < base + _TH)
        dl = jnp.where(inr, d - base, 0)
        mf = inr.astype(jnp.float32)
        asv = asad_ref[pl.ds(s, 1), 0:8]
        adv = asad_ref[pl.ds(d, 1), 8:16]
        ee = jnp.exp(_leaky(asv + adv, 0.2) - m1) * mf
        eerow = pltpu.repeat(ee, 16, axis=1)
        eemat = jnp.broadcast_to(eerow, (8, 128))
        row = h_ref[pl.ds(s * 8, 8), :]
        u_ref[pl.ds(dl * 8, 8), :] += row * eemat
        den_ref[pl.ds(dl, 1), :] += ee
        return 0

    lax.fori_loop(0, _CH, body, 0)


def _k3(u_ref, den_ref, b1_ref, w2_ref, as2v_ref, ad2v_ref,
        h2_ref, as2_ref, ad2_ref, asm_o, adm_o):
    i = pl.program_id(0)
    denrep = pltpu.repeat(den_ref[...], 128, axis=1)
    h1a = _leaky(u_ref[...] / denrep + b1_ref[...], 0.01)
    h2 = jnp.dot(h1a, w2_ref[...], preferred_element_type=jnp.float32, precision=jax.lax.Precision.HIGHEST)
    h2_ref[...] = h2
    a_s = jnp.dot(h2, as2v_ref[...], preferred_element_type=jnp.float32, precision=jax.lax.Precision.HIGHEST)
    a_d = jnp.dot(h2, ad2v_ref[...], preferred_element_type=jnp.float32, precision=jax.lax.Precision.HIGHEST)
    as2_ref[...] = a_s
    ad2_ref[...] = a_d
    bs = jnp.max(a_s, axis=0, keepdims=True)
    bd = jnp.max(a_d, axis=0, keepdims=True)

    @pl.when(i == 0)
    def _():
        asm_o[...] = bs
        adm_o[...] = bd

    @pl.when(i > 0)
    def _():
        asm_o[...] = jnp.maximum(asm_o[...], bs)
        adm_o[...] = jnp.maximum(adm_o[...], bd)


def _k4(src_ref, dst_ref, as2_ref, ad2_ref, m2_ref, ee2_ref, den2_ref):
    @pl.when(pl.program_id(0) == 0)
    def _():
        den2_ref[...] = jnp.zeros_like(den2_ref)

    m2 = m2_ref[...]

    def body(i, _):
        s = src_ref[0, 0, i]
        d = dst_ref[0, 0, i]
        ev = as2_ref[pl.ds(s, 1), :] + ad2_ref[pl.ds(d, 1), :]
        ee = jnp.exp(_leaky(ev, 0.2) - m2)
        ee2_ref[pl.ds(i, 1), :] = ee
        den2_ref[pl.ds(d, 1), :] += ee
        return 0

    lax.fori_loop(0, _CH, body, 0)


def _k5(src_ref, dst_ref, ee2_ref, den2_ref, w_ref):
    @pl.when(pl.program_id(0) == 0)
    def _():
        w_ref[...] = jnp.zeros_like(w_ref)

    def body(i, _):
        s = src_ref[0, 0, i]
        d = dst_ref[0, 0, i]
        a2 = ee2_ref[pl.ds(i, 1), :] / den2_ref[pl.ds(d, 1), :]
        w_ref[pl.ds(s, 1), :] += a2
        return 0

    lax.fori_loop(0, _CH, body, 0)


def _k6(h2_ref, w_ref, b2_ref, wl_ref, bl_ref, o_ref):
    v = h2_ref[...] * w_ref[...]
    ones = jnp.ones((1, v.shape[0]), jnp.float32)
    pooled = jnp.dot(ones, v, preferred_element_type=jnp.float32,
                     precision=jax.lax.Precision.HIGHEST)
    pooled = pooled + _N * b2_ref[...]
    o_ref[...] = jnp.dot(pooled, wl_ref[...],
                         preferred_element_type=jnp.float32, precision=jax.lax.Precision.HIGHEST) + bl_ref[...]


def kernel(x, edge_index, edge_type, edge_attr, W1, a_src1, a_dst1, b1, W2, a_src2, a_dst2, b2, Wl, bl):
    f32 = jnp.float32
    src0 = edge_index[0].astype(jnp.int32)
    dst0 = edge_index[1].astype(jnp.int32)
    loop = jnp.arange(_N, dtype=jnp.int32)
    src3 = jnp.concatenate([src0, loop]).reshape(_NB, 1, _CH)
    dst3 = jnp.concatenate([dst0, loop]).reshape(_NB, 1, _CH)

    j = jnp.arange(_H * _C)
    srccol = (j % _H) * _C + j // _H
    W1p = W1[:, srccol]
    b1i = b1[srccol].reshape(1, _H * _C)
    W2p = W2[srccol, :]
    asm = jnp.zeros((_H * _C, _H), f32).at[j, j % _H].set(a_src1[j % _H, j // _H])
    adm = jnp.zeros((_H * _C, _H), f32).at[j, j % _H].set(a_dst1[j % _H, j // _H])

    h1i, asad, asmax, admax = pl.pallas_call(
        _k1,
        grid=(_N // _BN,),
        in_specs=[
            pl.BlockSpec((_BN, _C), lambda i: (i, 0)),
            pl.BlockSpec((_C, _H * _C), lambda i: (0, 0)),
            pl.BlockSpec((_H * _C, _H), lambda i: (0, 0)),
            pl.BlockSpec((_H * _C, _H), lambda i: (0, 0)),
        ],
        out_specs=[
            pl.BlockSpec((_BN, _H * _C), lambda i: (i, 0)),
            pl.BlockSpec((_BN, 2 * _H), lambda i: (i, 0)),
            pl.BlockSpec((1, _H), lambda i: (0, 0)),
            pl.BlockSpec((1, _H), lambda i: (0, 0)),
        ],
        out_shape=[
            jax.ShapeDtypeStruct((_N, _H * _C), f32),
            jax.ShapeDtypeStruct((_N, 2 * _H), f32),
            jax.ShapeDtypeStruct((1, _H), f32),
            jax.ShapeDtypeStruct((1, _H), f32),
        ],
    )(x, W1p, asm, adm)
    M1 = _leaky(asmax + admax, 0.2)
    h1v = h1i.reshape(_N * _H, _C)

    def run_k2(t):
        return pl.pallas_call(
            functools.partial(_k2, t),
            grid=(_NB,),
            in_specs=[
                pl.BlockSpec((1, 1, _CH), lambda i: (i, 0, 0), memory_space=pltpu.SMEM),
                pl.BlockSpec((1, 1, _CH), lambda i: (i, 0, 0), memory_space=pltpu.SMEM),
                pl.BlockSpec((_N * _H, _C), lambda i: (0, 0)),
                pl.BlockSpec((_N, 2 * _H), lambda i: (0, 0)),
                pl.BlockSpec((1, _H), lambda i: (0, 0)),
            ],
            out_specs=[
                pl.BlockSpec((_TH * _H, _C), lambda i: (0, 0)),
                pl.BlockSpec((_TH, _H), lambda i: (0, 0)),
            ],
            out_shape=[
                jax.ShapeDtypeStruct((_TH * _H, _C), f32),
                jax.ShapeDtypeStruct((_TH, _H), f32),
            ],
            compiler_params=pltpu.CompilerParams(
                dimension_semantics=("arbitrary",),
                vmem_limit_bytes=_VMEM_LIM,
            ),
        )(src3, dst3, h1v, asad, M1)

    us, dens = zip(*[run_k2(t) for t in range(_NT)])
    U2 = jnp.concatenate(us, axis=0)[: _N * _H].reshape(_N, _H * _C)
    den = jnp.concatenate(dens, axis=0)[:_N]

    a_src2v = a_src2.reshape(_C, 1)
    a_dst2v = a_dst2.reshape(_C, 1)
    h2, AS2, AD2, as2max, ad2max = pl.pallas_call(
        _k3,
        grid=(_N // _BN,),
        in_specs=[
            pl.BlockSpec((_BN, _H * _C), lambda i: (i, 0)),
            pl.BlockSpec((_BN, _H), lambda i: (i, 0)),
            pl.BlockSpec((1, _H * _C), lambda i: (0, 0)),
            pl.BlockSpec((_H * _C, _C), lambda i: (0, 0)),
            pl.BlockSpec((_C, 1), lambda i: (0, 0)),
            pl.BlockSpec((_C, 1), lambda i: (0, 0)),
        ],
        out_specs=[
            pl.BlockSpec((_BN, _C), lambda i: (i, 0)),
            pl.BlockSpec((_BN, 1), lambda i: (i, 0)),
            pl.BlockSpec((_BN, 1), lambda i: (i, 0)),
            pl.BlockSpec((1, 1), lambda i: (0, 0)),
            pl.BlockSpec((1, 1), lambda i: (0, 0)),
        ],
        out_shape=[
            jax.ShapeDtypeStruct((_N, _C), f32),
            jax.ShapeDtypeStruct((_N, 1), f32),
            jax.ShapeDtypeStruct((_N, 1), f32),
            jax.ShapeDtypeStruct((1, 1), f32),
            jax.ShapeDtypeStruct((1, 1), f32),
        ],
    )(U2, den, b1i, W2p, a_src2v, a_dst2v)
    M2 = _leaky(as2max + ad2max, 0.2)

    ee2, den2 = pl.pallas_call(
        _k4,
        grid=(_NB,),
        in_specs=[
            pl.BlockSpec((1, 1, _CH), lambda i: (i, 0, 0), memory_space=pltpu.SMEM),
            pl.BlockSpec((1, 1, _CH), lambda i: (i, 0, 0), memory_space=pltpu.SMEM),
            pl.BlockSpec((_N, 1), lambda i: (0, 0)),
            pl.BlockSpec((_N, 1), lambda i: (0, 0)),
            pl.BlockSpec((1, 1), lambda i: (0, 0)),
        ],
        out_specs=[
            pl.BlockSpec((_CH, 1), lambda i: (i, 0)),
            pl.BlockSpec((_N, 1), lambda i: (0, 0)),
        ],
        out_shape=[
            jax.ShapeDtypeStruct((_EP, 1), f32),
            jax.ShapeDtypeStruct((_N, 1), f32),
        ],
        compiler_params=pltpu.CompilerParams(
            dimension_semantics=("arbitrary",),
        ),
    )(src3, dst3, AS2, AD2, M2)

    w = pl.pallas_call(
        _k5,
        grid=(_NB,),
        in_specs=[
            pl.BlockSpec((1, 1, _CH), lambda i: (i, 0, 0), memory_space=pltpu.SMEM),
            pl.BlockSpec((1, 1, _CH), lambda i: (i, 0, 0), memory_space=pltpu.SMEM),
            pl.BlockSpec((_CH, 1), lambda i: (i, 0)),
            pl.BlockSpec((_N, 1), lambda i: (0, 0)),
        ],
        out_specs=pl.BlockSpec((_N, 1), lambda i: (0, 0)),
        out_shape=jax.ShapeDtypeStruct((_N, 1), f32),
        compiler_params=pltpu.CompilerParams(
            dimension_semantics=("arbitrary",),
        ),
    )(src3, dst3, ee2, den2)

    out = pl.pallas_call(
        _k6,
        in_specs=[
            pl.BlockSpec((_N, _C), lambda: (0, 0)),
            pl.BlockSpec((_N, 1), lambda: (0, 0)),
            pl.BlockSpec((1, _C), lambda: (0, 0)),
            pl.BlockSpec((_C, 2), lambda: (0, 0)),
            pl.BlockSpec((1, 2), lambda: (0, 0)),
        ],
        out_specs=pl.BlockSpec((1, 2), lambda: (0, 0)),
        out_shape=jax.ShapeDtypeStruct((1, 2), f32),
    )(h2, w, b2.reshape(1, _C), Wl, bl.reshape(1, 2))
    return out


# speedup vs baseline: 2.7932x; 2.7932x over previous
import functools

import jax
import jax.numpy as jnp
from jax import lax
from jax.experimental import pallas as pl
from jax.experimental.pallas import tpu as pltpu

_N = 10000
_H = 8
_C = 128
_E = 320000
_EP = _E + _N
_CH = 3000
_NB = _EP // _CH
_BN = 1000
_TH = 3336
_NT = 3

_VMEM_LIM = 100 * 1024 * 1024


def _leaky(v, slope):
    return jnp.where(v >= 0, v, slope * v)


def _k1(x_ref, w_ref, asm_ref, adm_ref, h_ref, asad_ref, asm_o, adm_o):
    i = pl.program_id(0)
    h = jnp.dot(x_ref[...], w_ref[...], preferred_element_type=jnp.float32, precision=jax.lax.Precision.HIGHEST)
    h_ref[...] = h
    a_s = jnp.dot(h, asm_ref[...], preferred_element_type=jnp.float32, precision=jax.lax.Precision.HIGHEST)
    a_d = jnp.dot(h, adm_ref[...], preferred_element_type=jnp.float32, precision=jax.lax.Precision.HIGHEST)
    asad_ref[...] = jnp.concatenate([a_s, a_d], axis=1)
    bmax_s = jnp.max(a_s, axis=0, keepdims=True)
    bmax_d = jnp.max(a_d, axis=0, keepdims=True)

    @pl.when(i == 0)
    def _():
        asm_o[...] = bmax_s
        adm_o[...] = bmax_d

    @pl.when(i > 0)
    def _():
        asm_o[...] = jnp.maximum(asm_o[...], bmax_s)
        adm_o[...] = jnp.maximum(adm_o[...], bmax_d)


def _k2(t, src_ref, dst_ref, h_ref, asad_ref, m1_ref, u_ref, den_ref):
    @pl.when(pl.program_id(0) == 0)
    def _():
        u_ref[...] = jnp.zeros_like(u_ref)
        den_ref[...] = jnp.zeros_like(den_ref)

    m1 = m1_ref[...]
    base = t * _TH

    def edge(i):
        s = src_ref[0, 0, i]
        d = dst_ref[0, 0, i]
        inr = jnp.logical_and(d >= base, d < base + _TH)
        dl = jnp.where(inr, d - base, 0)
        mf = inr.astype(jnp.float32)
        asv = asad_ref[pl.ds(s, 1), 0:8]
        adv = asad_ref[pl.ds(d, 1), 8:16]
        ee = jnp.exp(_leaky(asv + adv, 0.2) - m1) * mf
        eerow = pltpu.repeat(ee, 16, axis=1)
        eemat = jnp.broadcast_to(eerow, (8, 128))
        row = h_ref[pl.ds(s * 8, 8), :]
        return dl, row * eemat, ee

    def body(i, _):
        dl0, c0, e0 = edge(4 * i)
        dl1, c1, e1 = edge(4 * i + 1)
        dl2, c2, e2 = edge(4 * i + 2)
        dl3, c3, e3 = edge(4 * i + 3)
        u_ref[pl.ds(dl0 * 8, 8), :] += c0
        den_ref[pl.ds(dl0, 1), :] += e0
        u_ref[pl.ds(dl1 * 8, 8), :] += c1
        den_ref[pl.ds(dl1, 1), :] += e1
        u_ref[pl.ds(dl2 * 8, 8), :] += c2
        den_ref[pl.ds(dl2, 1), :] += e2
        u_ref[pl.ds(dl3 * 8, 8), :] += c3
        den_ref[pl.ds(dl3, 1), :] += e3
        return 0

    lax.fori_loop(0, _CH // 4, body, 0)


def _k3(u_ref, den_ref, b1_ref, w2_ref, as2v_ref, ad2v_ref,
        h2_ref, as2_ref, ad2_ref, asm_o, adm_o):
    i = pl.program_id(0)
    denrep = pltpu.repeat(den_ref[...], 128, axis=1)
    h1a = _leaky(u_ref[...] / denrep + b1_ref[...], 0.01)
    h2 = jnp.dot(h1a, w2_ref[...], preferred_element_type=jnp.float32, precision=jax.lax.Precision.HIGHEST)
    h2_ref[...] = h2
    a_s = jnp.dot(h2, as2v_ref[...], preferred_element_type=jnp.float32, precision=jax.lax.Precision.HIGHEST)
    a_d = jnp.dot(h2, ad2v_ref[...], preferred_element_type=jnp.float32, precision=jax.lax.Precision.HIGHEST)
    as2_ref[...] = a_s
    ad2_ref[...] = a_d
    bs = jnp.max(a_s, axis=0, keepdims=True)
    bd = jnp.max(a_d, axis=0, keepdims=True)

    @pl.when(i == 0)
    def _():
        asm_o[...] = bs
        adm_o[...] = bd

    @pl.when(i > 0)
    def _():
        asm_o[...] = jnp.maximum(asm_o[...], bs)
        adm_o[...] = jnp.maximum(adm_o[...], bd)


def _k4(src_ref, dst_ref, as2_ref, ad2_ref, m2_ref, ee2_ref, den2_ref):
    @pl.when(pl.program_id(0) == 0)
    def _():
        den2_ref[...] = jnp.zeros_like(den2_ref)

    m2 = m2_ref[...]

    def body(i, _):
        s = src_ref[0, 0, i]
        d = dst_ref[0, 0, i]
        ev = as2_ref[pl.ds(s, 1), :] + ad2_ref[pl.ds(d, 1), :]
        ee = jnp.exp(_leaky(ev, 0.2) - m2)
        ee2_ref[pl.ds(i, 1), :] = ee
        den2_ref[pl.ds(d, 1), :] += ee
        return 0

    lax.fori_loop(0, _CH, body, 0)


def _k5(src_ref, dst_ref, ee2_ref, den2_ref, w_ref):
    @pl.when(pl.program_id(0) == 0)
    def _():
        w_ref[...] = jnp.zeros_like(w_ref)

    def body(i, _):
        s = src_ref[0, 0, i]
        d = dst_ref[0, 0, i]
        a2 = ee2_ref[pl.ds(i, 1), :] / den2_ref[pl.ds(d, 1), :]
        w_ref[pl.ds(s, 1), :] += a2
        return 0

    lax.fori_loop(0, _CH, body, 0)


def _k6(h2_ref, w_ref, b2_ref, wl_ref, bl_ref, o_ref):
    v = h2_ref[...] * w_ref[...]
    ones = jnp.ones((1, v.shape[0]), jnp.float32)
    pooled = jnp.dot(ones, v, preferred_element_type=jnp.float32,
                     precision=jax.lax.Precision.HIGHEST)
    pooled = pooled + _N * b2_ref[...]
    o_ref[...] = jnp.dot(pooled, wl_ref[...],
                         preferred_element_type=jnp.float32, precision=jax.lax.Precision.HIGHEST) + bl_ref[...]


def kernel(x, edge_index, edge_type, edge_attr, W1, a_src1, a_dst1, b1, W2, a_src2, a_dst2, b2, Wl, bl):
    f32 = jnp.float32
    src0 = edge_index[0].astype(jnp.int32)
    dst0 = edge_index[1].astype(jnp.int32)
    loop = jnp.arange(_N, dtype=jnp.int32)
    src3 = jnp.concatenate([src0, loop]).reshape(_NB, 1, _CH)
    dst3 = jnp.concatenate([dst0, loop]).reshape(_NB, 1, _CH)

    j = jnp.arange(_H * _C)
    srccol = (j % _H) * _C + j // _H
    W1p = W1[:, srccol]
    b1i = b1[srccol].reshape(1, _H * _C)
    W2p = W2[srccol, :]
    asm = jnp.zeros((_H * _C, _H), f32).at[j, j % _H].set(a_src1[j % _H, j // _H])
    adm = jnp.zeros((_H * _C, _H), f32).at[j, j % _H].set(a_dst1[j % _H, j // _H])

    h1i, asad, asmax, admax = pl.pallas_call(
        _k1,
        grid=(_N // _BN,),
        in_specs=[
            pl.BlockSpec((_BN, _C), lambda i: (i, 0)),
            pl.BlockSpec((_C, _H * _C), lambda i: (0, 0)),
            pl.BlockSpec((_H * _C, _H), lambda i: (0, 0)),
            pl.BlockSpec((_H * _C, _H), lambda i: (0, 0)),
        ],
        out_specs=[
            pl.BlockSpec((_BN, _H * _C), lambda i: (i, 0)),
            pl.BlockSpec((_BN, 2 * _H), lambda i: (i, 0)),
            pl.BlockSpec((1, _H), lambda i: (0, 0)),
            pl.BlockSpec((1, _H), lambda i: (0, 0)),
        ],
        out_shape=[
            jax.ShapeDtypeStruct((_N, _H * _C), f32),
            jax.ShapeDtypeStruct((_N, 2 * _H), f32),
            jax.ShapeDtypeStruct((1, _H), f32),
            jax.ShapeDtypeStruct((1, _H), f32),
        ],
    )(x, W1p, asm, adm)
    M1 = _leaky(asmax + admax, 0.2)
    h1v = h1i.reshape(_N * _H, _C)

    def run_k2(t):
        return pl.pallas_call(
            functools.partial(_k2, t),
            grid=(_NB,),
            in_specs=[
                pl.BlockSpec((1, 1, _CH), lambda i: (i, 0, 0), memory_space=pltpu.SMEM),
                pl.BlockSpec((1, 1, _CH), lambda i: (i, 0, 0), memory_space=pltpu.SMEM),
                pl.BlockSpec((_N * _H, _C), lambda i: (0, 0)),
                pl.BlockSpec((_N, 2 * _H), lambda i: (0, 0)),
                pl.BlockSpec((1, _H), lambda i: (0, 0)),
            ],
            out_specs=[
                pl.BlockSpec((_TH * _H, _C), lambda i: (0, 0)),
                pl.BlockSpec((_TH, _H), lambda i: (0, 0)),
            ],
            out_shape=[
                jax.ShapeDtypeStruct((_TH * _H, _C), f32),
                jax.ShapeDtypeStruct((_TH, _H), f32),
            ],
            compiler_params=pltpu.CompilerParams(
                dimension_semantics=("arbitrary",),
                vmem_limit_bytes=_VMEM_LIM,
            ),
        )(src3, dst3, h1v, asad, M1)

    us, dens = zip(*[run_k2(t) for t in range(_NT)])
    U2 = jnp.concatenate(us, axis=0)[: _N * _H].reshape(_N, _H * _C)
    den = jnp.concatenate(dens, axis=0)[:_N]

    a_src2v = a_src2.reshape(_C, 1)
    a_dst2v = a_dst2.reshape(_C, 1)
    h2, AS2, AD2, as2max, ad2max = pl.pallas_call(
        _k3,
        grid=(_N // _BN,),
        in_specs=[
            pl.BlockSpec((_BN, _H * _C), lambda i: (i, 0)),
            pl.BlockSpec((_BN, _H), lambda i: (i, 0)),
            pl.BlockSpec((1, _H * _C), lambda i: (0, 0)),
            pl.BlockSpec((_H * _C, _C), lambda i: (0, 0)),
            pl.BlockSpec((_C, 1), lambda i: (0, 0)),
            pl.BlockSpec((_C, 1), lambda i: (0, 0)),
        ],
        out_specs=[
            pl.BlockSpec((_BN, _C), lambda i: (i, 0)),
            pl.BlockSpec((_BN, 1), lambda i: (i, 0)),
            pl.BlockSpec((_BN, 1), lambda i: (i, 0)),
            pl.BlockSpec((1, 1), lambda i: (0, 0)),
            pl.BlockSpec((1, 1), lambda i: (0, 0)),
        ],
        out_shape=[
            jax.ShapeDtypeStruct((_N, _C), f32),
            jax.ShapeDtypeStruct((_N, 1), f32),
            jax.ShapeDtypeStruct((_N, 1), f32),
            jax.ShapeDtypeStruct((1, 1), f32),
            jax.ShapeDtypeStruct((1, 1), f32),
        ],
    )(U2, den, b1i, W2p, a_src2v, a_dst2v)
    M2 = _leaky(as2max + ad2max, 0.2)

    ee2, den2 = pl.pallas_call(
        _k4,
        grid=(_NB,),
        in_specs=[
            pl.BlockSpec((1, 1, _CH), lambda i: (i, 0, 0), memory_space=pltpu.SMEM),
            pl.BlockSpec((1, 1, _CH), lambda i: (i, 0, 0), memory_space=pltpu.SMEM),
            pl.BlockSpec((_N, 1), lambda i: (0, 0)),
            pl.BlockSpec((_N, 1), lambda i: (0, 0)),
            pl.BlockSpec((1, 1), lambda i: (0, 0)),
        ],
        out_specs=[
            pl.BlockSpec((_CH, 1), lambda i: (i, 0)),
            pl.BlockSpec((_N, 1), lambda i: (0, 0)),
        ],
        out_shape=[
            jax.ShapeDtypeStruct((_EP, 1), f32),
            jax.ShapeDtypeStruct((_N, 1), f32),
        ],
        compiler_params=pltpu.CompilerParams(
            dimension_semantics=("arbitrary",),
        ),
    )(src3, dst3, AS2, AD2, M2)

    w = pl.pallas_call(
        _k5,
        grid=(_NB,),
        in_specs=[
            pl.BlockSpec((1, 1, _CH), lambda i: (i, 0, 0), memory_space=pltpu.SMEM),
            pl.BlockSpec((1, 1, _CH), lambda i: (i, 0, 0), memory_space=pltpu.SMEM),
            pl.BlockSpec((_CH, 1), lambda i: (i, 0)),
            pl.BlockSpec((_N, 1), lambda i: (0, 0)),
        ],
        out_specs=pl.BlockSpec((_N, 1), lambda i: (0, 0)),
        out_shape=jax.ShapeDtypeStruct((_N, 1), f32),
        compiler_params=pltpu.CompilerParams(
            dimension_semantics=("arbitrary",),
        ),
    )(src3, dst3, ee2, den2)

    out = pl.pallas_call(
        _k6,
        in_specs=[
            pl.BlockSpec((_N, _C), lambda: (0, 0)),
            pl.BlockSpec((_N, 1), lambda: (0, 0)),
            pl.BlockSpec((1, _C), lambda: (0, 0)),
            pl.BlockSpec((_C, 2), lambda: (0, 0)),
            pl.BlockSpec((1, 2), lambda: (0, 0)),
        ],
        out_specs=pl.BlockSpec((1, 2), lambda: (0, 0)),
        out_shape=jax.ShapeDtypeStruct((1, 2), f32),
    )(h2, w, b2.reshape(1, _C), Wl, bl.reshape(1, 2))
    return out
